# baseline (device time: 205171 ns/iter reference)
import jax
import jax.numpy as jnp
from jax import lax
from jax.experimental import pallas as pl
from jax.experimental.pallas import tpu as pltpu

N_DEV = 8
M, N = 4096, 2048
CH = M // N_DEV
NRING = 4
QC = N // NRING
NSLOTS = 2
NSTEPS = 2 * (N_DEV - 1)


def kernel(x, w_mat, scale_x, scale_w):
    def body(x_ref, w_ref, sx_ref, sw_ref, out_ref,
             w_bf, stage, comm, sends, recvs, credits):
        my = lax.axis_index("i")
        left = lax.rem(my + N_DEV - 1, N_DEV)
        right = lax.rem(my + 1, N_DEV)

        def rows(c):
            return pl.ds(c * CH, CH)

        def ch(k):
            return lax.rem(my + k + 4 * N_DEV, N_DEV)

        rings = []
        for q, cwq in ((0, True), (2, False), (1, True), (3, False)):
            rings.append(dict(
                sg=-1 if cwq else 1,
                dst=right if cwq else left,
                credit_to=left if cwq else right,
                co=pl.ds(q * QC, QC)))

        w_bf[...] = w_ref[...].astype(jnp.bfloat16)

        def gemm(k):
            c = ch(k)
            out_ref[rows(c), :] = lax.dot_general(
                x_ref[rows(c), :].astype(jnp.bfloat16), w_bf[...],
                (((1,), (0,)), ((), ())),
                preferred_element_type=jnp.float32)

        gemm(0)

        barrier = pltpu.get_barrier_semaphore()
        for nbr in (left, right):
            pl.semaphore_signal(barrier, inc=1, device_id=(nbr,),
                                device_id_type=pl.DeviceIdType.MESH)
        pl.semaphore_wait(barrier, 2)

        def copy(src, dst, ssem, rsem, dev):
            return pltpu.make_async_remote_copy(
                src_ref=src, dst_ref=dst, send_sem=ssem, recv_sem=rsem,
                device_id=(dev,), device_id_type=pl.DeviceIdType.MESH)

        def hop_src(r, u):
            if u < N_DEV - 1:
                return stage.at[r, u % NSLOTS]
            if u == N_DEV - 1:
                return stage.at[r, (N_DEV - 1) % NSLOTS]
            return comm.at[r, (u - 1) % NSLOTS]

        def start(r, u):
            copy(hop_src(r, u), comm.at[r, u % NSLOTS],
                 sends.at[r, u], recvs.at[r, u], rings[r]["dst"]).start()

        def wait_hop(r, u):
            copy(hop_src(r, u), comm.at[r, u % NSLOTS],
                 sends.at[r, u], recvs.at[r, u], rings[r]["dst"]).wait()

        def signal_credit(r):
            pl.semaphore_signal(credits.at[r], inc=1,
                                device_id=(rings[r]["credit_to"],),
                                device_id_type=pl.DeviceIdType.MESH)

        for r, cfg in enumerate(rings):
            stage[r, 0] = out_ref[rows(ch(0)), cfg["co"]].astype(jnp.bfloat16)
            start(r, 0)
        gemm(-1)
        gemm(1)

        for s in range(N_DEV - 1):
            slot = s % NSLOTS
            for r, cfg in enumerate(rings):
                wait_hop(r, s)
                rc = ch(cfg["sg"] * (s + 1))
                val = out_ref[rows(rc), cfg["co"]] + comm[r, slot].astype(
                    jnp.float32)
                signal_credit(r)
                if s < N_DEV - 2:
                    if s + 1 >= NSLOTS:
                        pl.semaphore_wait(credits.at[r], 1)
                    stage[r, (s + 1) % NSLOTS] = val.astype(jnp.bfloat16)
                    start(r, s + 1)
                else:
                    scale = sx_ref[0] * sw_ref[0]
                    blk = jnp.maximum(val * scale, 0.0)
                    out_ref[rows(rc), cfg["co"]] = blk
                    pl.semaphore_wait(credits.at[r], 1)
                    stage[r, (N_DEV - 1) % NSLOTS] = blk.astype(jnp.bfloat16)
                    start(r, N_DEV - 1)
            if s == 0:
                gemm(-2)
                gemm(2)
            elif s == 1:
                gemm(-3)
                gemm(3)
            elif s == 2:
                gemm(4)

        for u in range(N_DEV - 1, 2 * N_DEV - 3):
            t = u - (N_DEV - 1)
            slot = u % NSLOTS
            for r, cfg in enumerate(rings):
                wait_hop(r, u)
                if u >= N_DEV:
                    signal_credit(r)
                pl.semaphore_wait(credits.at[r], 1)
                start(r, u + 1)
                g = ch(cfg["sg"] * t)
                out_ref[rows(g), cfg["co"]] = comm[r, slot].astype(jnp.float32)

        u_last = 2 * N_DEV - 3
        for r, cfg in enumerate(rings):
            wait_hop(r, u_last)
            g = ch(cfg["sg"] * (N_DEV - 2))
            out_ref[rows(g), cfg["co"]] = comm[r, u_last % NSLOTS].astype(
                jnp.float32)

    return pl.pallas_call(
        body,
        out_shape=jax.ShapeDtypeStruct((M, N), jnp.float32),
        in_specs=[
            pl.BlockSpec(memory_space=pltpu.VMEM),
            pl.BlockSpec(memory_space=pltpu.VMEM),
            pl.BlockSpec(memory_space=pltpu.SMEM),
            pl.BlockSpec(memory_space=pltpu.SMEM),
        ],
        out_specs=pl.BlockSpec(memory_space=pltpu.VMEM),
        scratch_shapes=[
            pltpu.VMEM((512, N), jnp.bfloat16),
            pltpu.VMEM((NRING, NSLOTS, CH, QC), jnp.bfloat16),
            pltpu.VMEM((NRING, NSLOTS, CH, QC), jnp.bfloat16),
            pltpu.SemaphoreType.DMA((NRING, NSTEPS)),
            pltpu.SemaphoreType.DMA((NRING, NSTEPS)),
            pltpu.SemaphoreType.REGULAR((NRING,)),
        ],
        compiler_params=pltpu.CompilerParams(
            collective_id=0, vmem_limit_bytes=64 * 1024 * 1024),
    )(x, w_mat, scale_x, scale_w)


# device time: 204700 ns/iter; 1.0023x vs baseline; 1.0023x over previous
import jax
import jax.numpy as jnp
from jax import lax
from jax.experimental import pallas as pl
from jax.experimental.pallas import tpu as pltpu

N_DEV = 8
M, N = 4096, 2048
CH = M // N_DEV
NRING = 4
QC = N // NRING
NSLOTS = 3
NSTEPS = 2 * (N_DEV - 1)


def kernel(x, w_mat, scale_x, scale_w):
    def body(x_ref, w_ref, sx_ref, sw_ref, out_ref,
             w_bf, stage, comm, sends, recvs, credits):
        my = lax.axis_index("i")
        left = lax.rem(my + N_DEV - 1, N_DEV)
        right = lax.rem(my + 1, N_DEV)

        def rows(c):
            return pl.ds(c * CH, CH)

        def ch(k):
            return lax.rem(my + k + 4 * N_DEV, N_DEV)

        rings = []
        for q, cwq in ((0, True), (2, False), (1, True), (3, False)):
            rings.append(dict(
                sg=-1 if cwq else 1,
                dst=right if cwq else left,
                credit_to=left if cwq else right,
                co=pl.ds(q * QC, QC)))

        w_bf[...] = w_ref[...].astype(jnp.bfloat16)

        def gemm(k):
            c = ch(k)
            out_ref[rows(c), :] = lax.dot_general(
                x_ref[rows(c), :].astype(jnp.bfloat16), w_bf[...],
                (((1,), (0,)), ((), ())),
                preferred_element_type=jnp.float32)

        gemm(0)

        barrier = pltpu.get_barrier_semaphore()
        for nbr in (left, right):
            pl.semaphore_signal(barrier, inc=1, device_id=(nbr,),
                                device_id_type=pl.DeviceIdType.MESH)
        pl.semaphore_wait(barrier, 2)

        def copy(src, dst, ssem, rsem, dev):
            return pltpu.make_async_remote_copy(
                src_ref=src, dst_ref=dst, send_sem=ssem, recv_sem=rsem,
                device_id=(dev,), device_id_type=pl.DeviceIdType.MESH)

        def hop_src(r, u):
            if u < N_DEV - 1:
                return stage.at[r, u % NSLOTS]
            if u == N_DEV - 1:
                return stage.at[r, (N_DEV - 1) % NSLOTS]
            return comm.at[r, (u - 1) % NSLOTS]

        def start(r, u):
            copy(hop_src(r, u), comm.at[r, u % NSLOTS],
                 sends.at[r, u], recvs.at[r, u], rings[r]["dst"]).start()

        def wait_hop(r, u):
            copy(hop_src(r, u), comm.at[r, u % NSLOTS],
                 sends.at[r, u], recvs.at[r, u], rings[r]["dst"]).wait()

        def signal_credit(r):
            pl.semaphore_signal(credits.at[r], inc=1,
                                device_id=(rings[r]["credit_to"],),
                                device_id_type=pl.DeviceIdType.MESH)

        for r, cfg in enumerate(rings):
            stage[r, 0] = out_ref[rows(ch(0)), cfg["co"]].astype(jnp.bfloat16)
            start(r, 0)
        gemm(-1)
        gemm(1)

        for s in range(N_DEV - 1):
            slot = s % NSLOTS
            for r, cfg in enumerate(rings):
                wait_hop(r, s)
                rc = ch(cfg["sg"] * (s + 1))
                val = out_ref[rows(rc), cfg["co"]] + comm[r, slot].astype(
                    jnp.float32)
                signal_credit(r)
                if s < N_DEV - 2:
                    if s + 1 >= NSLOTS:
                        pl.semaphore_wait(credits.at[r], 1)
                    stage[r, (s + 1) % NSLOTS] = val.astype(jnp.bfloat16)
                    start(r, s + 1)
                else:
                    scale = sx_ref[0] * sw_ref[0]
                    blk = jnp.maximum(val * scale, 0.0)
                    out_ref[rows(rc), cfg["co"]] = blk
                    pl.semaphore_wait(credits.at[r], 1)
                    stage[r, (N_DEV - 1) % NSLOTS] = blk.astype(jnp.bfloat16)
                    start(r, N_DEV - 1)
            if s == 0:
                gemm(-2)
                gemm(2)
            elif s == 1:
                gemm(-3)
                gemm(3)
            elif s == 2:
                gemm(4)

        for u in range(N_DEV - 1, 2 * N_DEV - 3):
            t = u - (N_DEV - 1)
            slot = u % NSLOTS
            for r, cfg in enumerate(rings):
                wait_hop(r, u)
                if u >= N_DEV and u <= 2 * N_DEV - 3 - (NSLOTS - 1):
                    signal_credit(r)
                pl.semaphore_wait(credits.at[r], 1)
                start(r, u + 1)
                g = ch(cfg["sg"] * t)
                out_ref[rows(g), cfg["co"]] = comm[r, slot].astype(jnp.float32)

        u_last = 2 * N_DEV - 3
        for r, cfg in enumerate(rings):
            wait_hop(r, u_last)
            g = ch(cfg["sg"] * (N_DEV - 2))
            out_ref[rows(g), cfg["co"]] = comm[r, u_last % NSLOTS].astype(
                jnp.float32)

    return pl.pallas_call(
        body,
        out_shape=jax.ShapeDtypeStruct((M, N), jnp.float32),
        in_specs=[
            pl.BlockSpec(memory_space=pltpu.VMEM),
            pl.BlockSpec(memory_space=pltpu.VMEM),
            pl.BlockSpec(memory_space=pltpu.SMEM),
            pl.BlockSpec(memory_space=pltpu.SMEM),
        ],
        out_specs=pl.BlockSpec(memory_space=pltpu.VMEM),
        scratch_shapes=[
            pltpu.VMEM((512, N), jnp.bfloat16),
            pltpu.VMEM((NRING, NSLOTS, CH, QC), jnp.bfloat16),
            pltpu.VMEM((NRING, NSLOTS, CH, QC), jnp.bfloat16),
            pltpu.SemaphoreType.DMA((NRING, NSTEPS)),
            pltpu.SemaphoreType.DMA((NRING, NSTEPS)),
            pltpu.SemaphoreType.REGULAR((NRING,)),
        ],
        compiler_params=pltpu.CompilerParams(
            collective_id=0, vmem_limit_bytes=64 * 1024 * 1024),
    )(x, w_mat, scale_x, scale_w)


# device time: 204111 ns/iter; 1.0052x vs baseline; 1.0029x over previous
import jax
import jax.numpy as jnp
from jax import lax
from jax.experimental import pallas as pl
from jax.experimental.pallas import tpu as pltpu

N_DEV = 8
M, N = 4096, 2048
CH = M // N_DEV
NRING = 4
QC = N // NRING
NSLOTS = 3
NSTEPS = 2 * (N_DEV - 1)


def kernel(x, w_mat, scale_x, scale_w):
    def body(x_ref, w_ref, sx_ref, sw_ref, out_ref,
             w_bf, stage, comm, sends, recvs, credits):
        my = lax.axis_index("i")

        def ham(m):
            return jnp.where(m < 4, m, 11 - m)

        pos = ham(my)
        left = ham(lax.rem(pos + N_DEV - 1, N_DEV))
        right = ham(lax.rem(pos + 1, N_DEV))

        def rows(c):
            return pl.ds(c * CH, CH)

        def ch(k):
            return lax.rem(pos + k + 4 * N_DEV, N_DEV)

        rings = []
        for q, cwq in ((0, True), (2, False), (1, True), (3, False)):
            rings.append(dict(
                sg=-1 if cwq else 1,
                dst=right if cwq else left,
                credit_to=left if cwq else right,
                co=pl.ds(q * QC, QC)))

        w_bf[...] = w_ref[...].astype(jnp.bfloat16)

        def gemm(k):
            c = ch(k)
            out_ref[rows(c), :] = lax.dot_general(
                x_ref[rows(c), :].astype(jnp.bfloat16), w_bf[...],
                (((1,), (0,)), ((), ())),
                preferred_element_type=jnp.float32)

        gemm(0)

        barrier = pltpu.get_barrier_semaphore()
        for nbr in (left, right):
            pl.semaphore_signal(barrier, inc=1, device_id=(nbr,),
                                device_id_type=pl.DeviceIdType.MESH)
        pl.semaphore_wait(barrier, 2)

        def copy(src, dst, ssem, rsem, dev):
            return pltpu.make_async_remote_copy(
                src_ref=src, dst_ref=dst, send_sem=ssem, recv_sem=rsem,
                device_id=(dev,), device_id_type=pl.DeviceIdType.MESH)

        def hop_src(r, u):
            if u < N_DEV - 1:
                return stage.at[r, u % NSLOTS]
            if u == N_DEV - 1:
                return stage.at[r, (N_DEV - 1) % NSLOTS]
            return comm.at[r, (u - 1) % NSLOTS]

        def start(r, u):
            copy(hop_src(r, u), comm.at[r, u % NSLOTS],
                 sends.at[r, u], recvs.at[r, u], rings[r]["dst"]).start()

        def wait_hop(r, u):
            copy(hop_src(r, u), comm.at[r, u % NSLOTS],
                 sends.at[r, u], recvs.at[r, u], rings[r]["dst"]).wait()

        def signal_credit(r):
            pl.semaphore_signal(credits.at[r], inc=1,
                                device_id=(rings[r]["credit_to"],),
                                device_id_type=pl.DeviceIdType.MESH)

        for r, cfg in enumerate(rings):
            stage[r, 0] = out_ref[rows(ch(0)), cfg["co"]].astype(jnp.bfloat16)
            start(r, 0)
        gemm(-1)
        gemm(1)

        for s in range(N_DEV - 1):
            slot = s % NSLOTS
            for r, cfg in enumerate(rings):
                wait_hop(r, s)
                rc = ch(cfg["sg"] * (s + 1))
                val = out_ref[rows(rc), cfg["co"]] + comm[r, slot].astype(
                    jnp.float32)
                signal_credit(r)
                if s < N_DEV - 2:
                    if s + 1 >= NSLOTS:
                        pl.semaphore_wait(credits.at[r], 1)
                    stage[r, (s + 1) % NSLOTS] = val.astype(jnp.bfloat16)
                    start(r, s + 1)
                else:
                    scale = sx_ref[0] * sw_ref[0]
                    blk = jnp.maximum(val * scale, 0.0)
                    out_ref[rows(rc), cfg["co"]] = blk
                    pl.semaphore_wait(credits.at[r], 1)
                    stage[r, (N_DEV - 1) % NSLOTS] = blk.astype(jnp.bfloat16)
                    start(r, N_DEV - 1)
            if s == 0:
                gemm(-2)
                gemm(2)
            elif s == 1:
                gemm(-3)
                gemm(3)
            elif s == 2:
                gemm(4)

        for u in range(N_DEV - 1, 2 * N_DEV - 3):
            t = u - (N_DEV - 1)
            slot = u % NSLOTS
            for r, cfg in enumerate(rings):
                wait_hop(r, u)
                if u >= N_DEV and u <= 2 * N_DEV - 3 - (NSLOTS - 1):
                    signal_credit(r)
                pl.semaphore_wait(credits.at[r], 1)
                start(r, u + 1)
                g = ch(cfg["sg"] * t)
                out_ref[rows(g), cfg["co"]] = comm[r, slot].astype(jnp.float32)

        u_last = 2 * N_DEV - 3
        for r, cfg in enumerate(rings):
            wait_hop(r, u_last)
            g = ch(cfg["sg"] * (N_DEV - 2))
            out_ref[rows(g), cfg["co"]] = comm[r, u_last % NSLOTS].astype(
                jnp.float32)

    return pl.pallas_call(
        body,
        out_shape=jax.ShapeDtypeStruct((M, N), jnp.float32),
        in_specs=[
            pl.BlockSpec(memory_space=pltpu.VMEM),
            pl.BlockSpec(memory_space=pltpu.VMEM),
            pl.BlockSpec(memory_space=pltpu.SMEM),
            pl.BlockSpec(memory_space=pltpu.SMEM),
        ],
        out_specs=pl.BlockSpec(memory_space=pltpu.VMEM),
        scratch_shapes=[
            pltpu.VMEM((512, N), jnp.bfloat16),
            pltpu.VMEM((NRING, NSLOTS, CH, QC), jnp.bfloat16),
            pltpu.VMEM((NRING, NSLOTS, CH, QC), jnp.bfloat16),
            pltpu.SemaphoreType.DMA((NRING, NSTEPS)),
            pltpu.SemaphoreType.DMA((NRING, NSTEPS)),
            pltpu.SemaphoreType.REGULAR((NRING,)),
        ],
        compiler_params=pltpu.CompilerParams(
            collective_id=0, vmem_limit_bytes=64 * 1024 * 1024),
    )(x, w_mat, scale_x, scale_w)
